# baseline (speedup 1.0000x reference)
import jax
import jax.numpy as jnp
from jax.experimental import pallas as pl
from jax.experimental.pallas import tpu as pltpu


def _gcn_kernel(x_ref, wgc_ref, wfc_ref, bgc_ref, bfc_ref, fa0_ref, fa1_ref,
                out_ref, sup_ref):
    @pl.when(pl.program_id(0) == 0)
    def _():
        s = jnp.dot(x_ref[...].astype(jnp.bfloat16),
                    wgc_ref[...].astype(jnp.bfloat16),
                    preferred_element_type=jnp.float32)
        sup_ref[...] = s.astype(jnp.bfloat16)

    for k, fa_ref in enumerate((fa0_ref, fa1_ref)):
        a = fa_ref[...].astype(jnp.bfloat16)
        h = jnp.dot(a, sup_ref[...], preferred_element_type=jnp.float32)
        h = h + bgc_ref[...]
        h = jnp.where(h > 0, h, jnp.exp(jnp.minimum(h, 0.0)) - 1.0)
        out_ref[k * 200:(k + 1) * 200, :] = (
            jnp.dot(h, wfc_ref[...], preferred_element_type=jnp.float32)
            + bfc_ref[...]
        )


@jax.jit
def kernel(input, fadj, W_gc, b_gc, W_fc, b_fc):
    n, n_in = input.shape
    nfea = W_gc.shape[1]
    n_class = W_fc.shape[1]

    bm = 400
    half = bm // 2
    out = pl.pallas_call(
        _gcn_kernel,
        grid=(n // bm,),
        in_specs=[
            pl.BlockSpec((n, n_in), lambda i: (0, 0)),
            pl.BlockSpec((n_in, nfea), lambda i: (0, 0)),
            pl.BlockSpec((nfea, n_class), lambda i: (0, 0)),
            pl.BlockSpec((1, nfea), lambda i: (0, 0)),
            pl.BlockSpec((1, n_class), lambda i: (0, 0)),
            pl.BlockSpec((half, n), lambda i: (2 * i, 0)),
            pl.BlockSpec((half, n), lambda i: (2 * i + 1, 0)),
        ],
        out_specs=pl.BlockSpec((bm, n_class), lambda i: (i, 0)),
        out_shape=jax.ShapeDtypeStruct((n, n_class), jnp.float32),
        scratch_shapes=[pltpu.VMEM((n, nfea), jnp.bfloat16)],
        compiler_params=pltpu.CompilerParams(
            dimension_semantics=("arbitrary",),
        ),
    )(
        input,
        W_gc,
        W_fc,
        b_gc.reshape(1, nfea),
        b_fc.reshape(1, n_class),
        fadj,
        fadj,
    )

    return out


# speedup vs baseline: 1.1351x; 1.1351x over previous
import jax
import jax.numpy as jnp
from jax.experimental import pallas as pl
from jax.experimental.pallas import tpu as pltpu


def _gcn_kernel(x_ref, wgc_ref, wfc_ref, bgc_ref, bfc_ref, fadj_ref,
                out_ref, xb_ref):
    @pl.when(pl.program_id(0) == 0)
    def _():
        xb_ref[...] = x_ref[...].astype(jnp.bfloat16)

    a = fadj_ref[...].astype(jnp.bfloat16)
    t = jnp.dot(a, xb_ref[...], preferred_element_type=jnp.float32)
    h = jnp.dot(t.astype(jnp.bfloat16), wgc_ref[...].astype(jnp.bfloat16),
                preferred_element_type=jnp.float32)
    h = h + bgc_ref[...]
    h = jnp.where(h > 0, h, jnp.exp(jnp.minimum(h, 0.0)) - 1.0)
    out_ref[...] = (
        jnp.dot(h, wfc_ref[...], preferred_element_type=jnp.float32)
        + bfc_ref[...]
    )


@jax.jit
def kernel(input, fadj, W_gc, b_gc, W_fc, b_fc):
    n, n_in = input.shape
    nfea = W_gc.shape[1]
    n_class = W_fc.shape[1]

    bm = 400
    out = pl.pallas_call(
        _gcn_kernel,
        grid=(n // bm,),
        in_specs=[
            pl.BlockSpec((n, n_in), lambda i: (0, 0)),
            pl.BlockSpec((n_in, nfea), lambda i: (0, 0)),
            pl.BlockSpec((nfea, n_class), lambda i: (0, 0)),
            pl.BlockSpec((1, nfea), lambda i: (0, 0)),
            pl.BlockSpec((1, n_class), lambda i: (0, 0)),
            pl.BlockSpec((bm, n), lambda i: (i, 0)),
        ],
        out_specs=pl.BlockSpec((bm, n_class), lambda i: (i, 0)),
        out_shape=jax.ShapeDtypeStruct((n, n_class), jnp.float32),
        scratch_shapes=[pltpu.VMEM((n, n_in), jnp.bfloat16)],
        compiler_params=pltpu.CompilerParams(
            dimension_semantics=("arbitrary",),
        ),
    )(
        input,
        W_gc,
        W_fc,
        b_gc.reshape(1, nfea),
        b_fc.reshape(1, n_class),
        fadj,
    )

    return out
